# initial kernel scaffold (unmeasured)
import jax
import jax.numpy as jnp
from jax import lax
from jax.experimental import pallas as pl
from jax.experimental.pallas import tpu as pltpu

N_DEV = 32
EPS = 1e-5
LANES = 128


def kernel(x, gamma):
    m, n_per = x.shape
    n_global = n_per * N_DEV
    sub = m // LANES
    gamma2 = gamma.reshape(1, n_per)

    def body(x_ref, g_ref, out_ref, acc_ref, send_sems, recv_sems):
        my = lax.axis_index("i")

        xv = x_ref[:, :]
        local = jnp.sum(xv * xv, axis=1, keepdims=True)
        acc_ref[0, :, :] = local.reshape(sub, LANES)

        rdmas = []
        for d in range(1, N_DEV):
            dst = (my + d) % N_DEV
            rdma = pltpu.make_async_remote_copy(
                src_ref=acc_ref.at[0],
                dst_ref=acc_ref.at[d],
                send_sem=send_sems.at[d],
                recv_sem=recv_sems.at[d],
                device_id=(dst,),
                device_id_type=pl.DeviceIdType.MESH,
            )
            rdma.start()
            rdmas.append(rdma)

        for rdma in rdmas:
            rdma.wait_recv()
        for rdma in rdmas:
            rdma.wait_send()

        total = jnp.sum(acc_ref[:, :, :], axis=0)
        rinv = lax.rsqrt(total / n_global + EPS)
        rinv_col = rinv.reshape(m, 1)
        out_ref[:, :] = xv * g_ref[0:1, :] * rinv_col

    return pl.pallas_call(
        body,
        out_shape=jax.ShapeDtypeStruct((m, n_per), jnp.float32),
        in_specs=[
            pl.BlockSpec(memory_space=pltpu.VMEM),
            pl.BlockSpec(memory_space=pltpu.VMEM),
        ],
        out_specs=pl.BlockSpec(memory_space=pltpu.VMEM),
        scratch_shapes=[
            pltpu.VMEM((N_DEV, sub, LANES), jnp.float32),
            pltpu.SemaphoreType.DMA((N_DEV,)),
            pltpu.SemaphoreType.DMA((N_DEV,)),
        ],
        compiler_params=pltpu.CompilerParams(collective_id=0),
    )(x, gamma2)


# baseline (device time: 23769 ns/iter reference)
import jax
import jax.numpy as jnp
from jax import lax
from jax.experimental import pallas as pl
from jax.experimental.pallas import tpu as pltpu

N_DEV = 32
EPS = 1e-5
LANES = 128


def kernel(x, gamma):
    m, n_per = x.shape
    n_global = n_per * N_DEV
    sub = m // LANES
    gamma2 = gamma.reshape(1, n_per)

    def body(x_ref, g_ref, out_ref, acc_ref, send_sems, recv_sems):
        my = lax.axis_index("i")

        ri = lax.broadcasted_iota(jnp.int32, (m, LANES), 0)
        li = lax.broadcasted_iota(jnp.int32, (m, LANES), 1)
        Q = (ri % LANES == li).astype(jnp.float32)
        r2 = lax.broadcasted_iota(jnp.int32, (m, sub), 0)
        s2 = lax.broadcasted_iota(jnp.int32, (m, sub), 1)
        P = (r2 // LANES == s2).astype(jnp.float32)

        xv = x_ref[:, :]
        local = jnp.sum(xv * xv, axis=1, keepdims=True)
        acc_ref[0, :, :] = jax.lax.dot_general(
            P, local * Q, (((0,), (0,)), ((), ())),
            precision=lax.Precision.HIGHEST,
        )

        rdmas = []
        for d in range(1, N_DEV):
            dst = (my + d) % N_DEV
            rdma = pltpu.make_async_remote_copy(
                src_ref=acc_ref.at[0],
                dst_ref=acc_ref.at[d],
                send_sem=send_sems.at[d],
                recv_sem=recv_sems.at[d],
                device_id=(dst,),
                device_id_type=pl.DeviceIdType.MESH,
            )
            rdma.start()
            rdmas.append(rdma)

        for rdma in rdmas:
            rdma.wait_recv()
        for rdma in rdmas:
            rdma.wait_send()

        total = jnp.sum(acc_ref[:, :, :], axis=0)
        spread = jnp.dot(P, total, precision=lax.Precision.HIGHEST)
        tot_col = jnp.sum(spread * Q, axis=1, keepdims=True)
        rinv_col = lax.rsqrt(tot_col / n_global + EPS)
        out_ref[:, :] = xv * g_ref[0:1, :] * rinv_col

    return pl.pallas_call(
        body,
        out_shape=jax.ShapeDtypeStruct((m, n_per), jnp.float32),
        in_specs=[
            pl.BlockSpec(memory_space=pltpu.VMEM),
            pl.BlockSpec(memory_space=pltpu.VMEM),
        ],
        out_specs=pl.BlockSpec(memory_space=pltpu.VMEM),
        scratch_shapes=[
            pltpu.VMEM((N_DEV, sub, LANES), jnp.float32),
            pltpu.SemaphoreType.DMA((N_DEV,)),
            pltpu.SemaphoreType.DMA((N_DEV,)),
        ],
    )(x, gamma2)


# device time: 15961 ns/iter; 1.4892x vs baseline; 1.4892x over previous
import jax
import jax.numpy as jnp
from jax import lax
from jax.experimental import pallas as pl
from jax.experimental.pallas import tpu as pltpu

N_DEV = 32
EPS = 1e-5
LANES = 128


def kernel(x, gamma):
    m, n_per = x.shape
    n_global = n_per * N_DEV
    sub = m // LANES
    gamma2 = gamma.reshape(1, n_per)

    def body(x_ref, g_ref, out_ref, acc_ref, send_sems, recv_sems):
        my = lax.axis_index("i")

        barrier_sem = pltpu.get_barrier_semaphore()
        for d in range(1, N_DEV):
            pl.semaphore_signal(
                barrier_sem, inc=1,
                device_id=((my + d) % N_DEV,),
                device_id_type=pl.DeviceIdType.MESH,
            )

        ri = lax.broadcasted_iota(jnp.int32, (m, LANES), 0)
        li = lax.broadcasted_iota(jnp.int32, (m, LANES), 1)
        Q = (ri % LANES == li).astype(jnp.float32)
        r2 = lax.broadcasted_iota(jnp.int32, (m, sub), 0)
        s2 = lax.broadcasted_iota(jnp.int32, (m, sub), 1)
        P = (r2 // LANES == s2).astype(jnp.float32)

        xv = x_ref[:, :]
        local = jnp.sum(xv * xv, axis=1, keepdims=True)
        acc_ref[0, :, :] = jax.lax.dot_general(
            P, local * Q, (((0,), (0,)), ((), ())),
            precision=lax.Precision.HIGHEST,
        )

        pl.semaphore_wait(barrier_sem, N_DEV - 1)

        rdmas = []
        for d in range(1, N_DEV):
            dst = (my + d) % N_DEV
            rdma = pltpu.make_async_remote_copy(
                src_ref=acc_ref.at[0],
                dst_ref=acc_ref.at[d],
                send_sem=send_sems.at[d],
                recv_sem=recv_sems.at[d],
                device_id=(dst,),
                device_id_type=pl.DeviceIdType.MESH,
            )
            rdma.start()
            rdmas.append(rdma)

        for rdma in rdmas:
            rdma.wait_recv()
        for rdma in rdmas:
            rdma.wait_send()

        total = jnp.sum(acc_ref[:, :, :], axis=0)
        spread = jnp.dot(P, total, precision=lax.Precision.HIGHEST)
        tot_col = jnp.sum(spread * Q, axis=1, keepdims=True)
        rinv_col = lax.rsqrt(tot_col / n_global + EPS)
        out_ref[:, :] = xv * g_ref[0:1, :] * rinv_col

    return pl.pallas_call(
        body,
        out_shape=jax.ShapeDtypeStruct((m, n_per), jnp.float32),
        in_specs=[
            pl.BlockSpec(memory_space=pltpu.VMEM),
            pl.BlockSpec(memory_space=pltpu.VMEM),
        ],
        out_specs=pl.BlockSpec(memory_space=pltpu.VMEM),
        scratch_shapes=[
            pltpu.VMEM((N_DEV, sub, LANES), jnp.float32),
            pltpu.SemaphoreType.DMA((N_DEV,)),
            pltpu.SemaphoreType.DMA((N_DEV,)),
        ],
        compiler_params=pltpu.CompilerParams(collective_id=0),
    )(x, gamma2)


# device time: 15919 ns/iter; 1.4931x vs baseline; 1.0026x over previous
import jax
import jax.numpy as jnp
from jax import lax
from jax.experimental import pallas as pl
from jax.experimental.pallas import tpu as pltpu

N_DEV = 32
EPS = 1e-5
LANES = 128


def kernel(x, gamma):
    m, n_per = x.shape
    n_global = n_per * N_DEV
    sub = m // LANES
    gamma2 = gamma.reshape(1, n_per)

    def body(x_ref, g_ref, out_ref, acc_ref, send_sems, recv_sems):
        my = lax.axis_index("i")

        barrier_sem = pltpu.get_barrier_semaphore()
        for d in range(1, N_DEV):
            pl.semaphore_signal(
                barrier_sem, inc=1,
                device_id=((my + d) % N_DEV,),
                device_id_type=pl.DeviceIdType.MESH,
            )

        ri = lax.broadcasted_iota(jnp.int32, (m, LANES), 0)
        li = lax.broadcasted_iota(jnp.int32, (m, LANES), 1)
        Q = (ri % LANES == li).astype(jnp.float32)
        r2 = lax.broadcasted_iota(jnp.int32, (m, sub), 0)
        s2 = lax.broadcasted_iota(jnp.int32, (m, sub), 1)
        P = (r2 // LANES == s2).astype(jnp.float32)

        xv = x_ref[:, :]
        local = jnp.sum(xv * xv, axis=1, keepdims=True)
        acc_ref[0, :, :] = jax.lax.dot_general(
            P, local * Q, (((0,), (0,)), ((), ())),
            precision=lax.Precision.HIGHEST,
        )

        pl.semaphore_wait(barrier_sem, N_DEV - 1)

        rdmas = []
        for d in range(1, N_DEV):
            dst = (my + d) % N_DEV
            rdma = pltpu.make_async_remote_copy(
                src_ref=acc_ref.at[0],
                dst_ref=acc_ref.at[d],
                send_sem=send_sems.at[d],
                recv_sem=recv_sems.at[d],
                device_id=(dst,),
                device_id_type=pl.DeviceIdType.MESH,
            )
            rdma.start()
            rdmas.append(rdma)

        xg = xv * g_ref[0:1, :]

        for rdma in rdmas:
            rdma.wait_recv()
        for rdma in rdmas:
            rdma.wait_send()

        total = jnp.sum(acc_ref[:, :, :], axis=0)
        spread = jnp.dot(P, total, precision=lax.Precision.HIGHEST)
        tot_col = jnp.sum(spread * Q, axis=1, keepdims=True)
        rinv_col = lax.rsqrt(tot_col / n_global + EPS)
        out_ref[:, :] = xg * rinv_col

    return pl.pallas_call(
        body,
        out_shape=jax.ShapeDtypeStruct((m, n_per), jnp.float32),
        in_specs=[
            pl.BlockSpec(memory_space=pltpu.VMEM),
            pl.BlockSpec(memory_space=pltpu.VMEM),
        ],
        out_specs=pl.BlockSpec(memory_space=pltpu.VMEM),
        scratch_shapes=[
            pltpu.VMEM((N_DEV, sub, LANES), jnp.float32),
            pltpu.SemaphoreType.DMA((N_DEV,)),
            pltpu.SemaphoreType.DMA((N_DEV,)),
        ],
        compiler_params=pltpu.CompilerParams(collective_id=0),
    )(x, gamma2)
